# baseline (speedup 1.0000x reference)
import jax
import jax.numpy as jnp
from jax import lax
from jax.experimental import pallas as pl

_ROWS_PER_BLOCK = 16


def _sampler_block(x_ref, ids_ref, probs_ref):
    x = x_ref[...]
    r, v = x.shape
    m = jnp.max(x, axis=-1, keepdims=True)
    col = lax.broadcasted_iota(jnp.int32, (r, v), 1)
    idx = jnp.min(jnp.where(x == m, col, v), axis=-1)
    e = jnp.exp(x - m)
    s = jnp.sum(e, axis=-1, keepdims=True)
    probs_ref[...] = e * (1.0 / s)
    ids_ref[...] = idx[:, None]


def kernel(logits):
    n, v = logits.shape
    r = _ROWS_PER_BLOCK
    grid = (n // r,)
    ids, probs = pl.pallas_call(
        _sampler_block,
        grid=grid,
        in_specs=[pl.BlockSpec((r, v), lambda i: (i, 0))],
        out_specs=[
            pl.BlockSpec((r, 1), lambda i: (i, 0)),
            pl.BlockSpec((r, v), lambda i: (i, 0)),
        ],
        out_shape=[
            jax.ShapeDtypeStruct((n, 1), jnp.int32),
            jax.ShapeDtypeStruct((n, v), jnp.float32),
        ],
    )(logits)
    return (ids.reshape(n), probs)


# speedup vs baseline: 1.8105x; 1.8105x over previous
import jax
import jax.numpy as jnp
from jax import lax
from jax.experimental import pallas as pl
from jax.experimental.pallas import tpu as pltpu

_CHUNK = 10000


def _sampler_body(x_ref, probs_ref, ids_ref, m_ref, s_ref, a_ref):
    p = pl.program_id(0)
    i = pl.program_id(1)
    nsteps = pl.num_programs(1)
    c, n = x_ref.shape
    v = c * nsteps

    @pl.when(jnp.logical_and(p == 0, i == 0))
    def _init():
        m_ref[...] = jnp.full((1, n), -jnp.inf, jnp.float32)
        s_ref[...] = jnp.zeros((1, n), jnp.float32)
        a_ref[...] = jnp.zeros((1, n), jnp.int32)

    @pl.when(p == 0)
    def _stats():
        x = x_ref[...]
        run_m = m_ref[...]
        cmax = jnp.max(x, axis=0, keepdims=True)
        row = lax.broadcasted_iota(jnp.int32, (c, n), 0) + i * c
        carg = jnp.min(jnp.where(x == cmax, row, v), axis=0, keepdims=True)
        nmax = jnp.maximum(run_m, cmax)
        csum = jnp.sum(jnp.exp(x - nmax), axis=0, keepdims=True)
        s_ref[...] = s_ref[...] * jnp.exp(run_m - nmax) + csum
        a_ref[...] = jnp.where(cmax > run_m, carg, a_ref[...])
        m_ref[...] = nmax

    @pl.when(p == 1)
    def _write():
        x = x_ref[...]
        probs_ref[...] = jnp.exp(x - m_ref[...]) * (1.0 / s_ref[...])

        @pl.when(i == nsteps - 1)
        def _ids():
            ids_ref[...] = a_ref[...]


def kernel(logits):
    n, v = logits.shape
    c = _CHUNK
    nsteps = v // c
    x_t = logits.T
    probs_t, ids = pl.pallas_call(
        _sampler_body,
        grid=(2, nsteps),
        in_specs=[pl.BlockSpec((c, n), lambda p, i: (i, 0))],
        out_specs=[
            pl.BlockSpec((c, n), lambda p, i: (i * p, 0)),
            pl.BlockSpec((1, n), lambda p, i: (0, 0)),
        ],
        out_shape=[
            jax.ShapeDtypeStruct((v, n), jnp.float32),
            jax.ShapeDtypeStruct((1, n), jnp.int32),
        ],
        scratch_shapes=[
            pltpu.VMEM((1, n), jnp.float32),
            pltpu.VMEM((1, n), jnp.float32),
            pltpu.VMEM((1, n), jnp.int32),
        ],
    )(x_t)
    return (ids.reshape(n), probs_t.T)


# speedup vs baseline: 2.3389x; 1.2919x over previous
import jax
import jax.numpy as jnp
from jax import lax
from jax.experimental import pallas as pl
from jax.experimental.pallas import tpu as pltpu

_CHUNK = 10000
_CHAINS = 10
_LOG2E = 1.4426950408889634


def _sampler_body(x_ref, probs_ref, ids_ref, m_ref, s_ref, a_ref):
    p = pl.program_id(0)
    i = pl.program_id(1)
    nsteps = pl.num_programs(1)
    c, n = x_ref.shape
    v = c * nsteps
    k = _CHAINS
    d = c // k

    @pl.when(jnp.logical_and(p == 0, i == 0))
    def _init():
        m_ref[...] = jnp.full((1, n), -jnp.inf, jnp.float32)
        s_ref[...] = jnp.zeros((1, n), jnp.float32)
        a_ref[...] = jnp.zeros((1, n), jnp.int32)

    @pl.when(p == 0)
    def _stats():
        xr = x_ref[...].reshape(k, d, n)
        run_m = m_ref[...]
        pm = jnp.max(xr, axis=1)
        cmax = jnp.max(pm, axis=0, keepdims=True)
        row = lax.broadcasted_iota(jnp.int32, (k, d, n), 1)
        parg = jnp.min(jnp.where(xr == cmax[None], row, v), axis=1)
        offs = lax.broadcasted_iota(jnp.int32, (k, 1), 0) * d
        carg = jnp.min(parg + offs, axis=0, keepdims=True) + i * c
        nmax = jnp.maximum(run_m, cmax)
        bm = nmax * _LOG2E
        ps = jnp.sum(jnp.exp2(xr * _LOG2E - bm[None]), axis=1)
        csum = jnp.sum(ps, axis=0, keepdims=True)
        s_ref[...] = s_ref[...] * jnp.exp2(run_m * _LOG2E - bm) + csum
        a_ref[...] = jnp.where(cmax > run_m, carg, a_ref[...])
        m_ref[...] = nmax

    @pl.when(p == 1)
    def _write():
        x = x_ref[...]
        b = m_ref[...] * _LOG2E + jnp.log2(s_ref[...])
        probs_ref[...] = jnp.exp2(x * _LOG2E - b)

        @pl.when(i == nsteps - 1)
        def _ids():
            ids_ref[...] = a_ref[...]


def kernel(logits):
    n, v = logits.shape
    c = _CHUNK
    nsteps = v // c
    x_t = logits.T
    probs_t, ids = pl.pallas_call(
        _sampler_body,
        grid=(2, nsteps),
        in_specs=[pl.BlockSpec((c, n), lambda p, i: (i, 0))],
        out_specs=[
            pl.BlockSpec((c, n), lambda p, i: (i * p, 0)),
            pl.BlockSpec((1, n), lambda p, i: (0, 0)),
        ],
        out_shape=[
            jax.ShapeDtypeStruct((v, n), jnp.float32),
            jax.ShapeDtypeStruct((1, n), jnp.int32),
        ],
        scratch_shapes=[
            pltpu.VMEM((1, n), jnp.float32),
            pltpu.VMEM((1, n), jnp.float32),
            pltpu.VMEM((1, n), jnp.int32),
        ],
    )(x_t)
    return (ids.reshape(n), probs_t.T)


# speedup vs baseline: 3.1260x; 1.3365x over previous
import jax
import jax.numpy as jnp
from jax import lax
from jax.experimental import pallas as pl
from jax.experimental.pallas import tpu as pltpu

_CHUNK = 10000
_CHAINS = 10
_LOG2E = 1.4426950408889634


def _slab_copy(x_hbm, scr, sem, i, c):
    return pltpu.make_async_copy(
        x_hbm.at[pl.ds(i * c, c), :], scr.at[pl.ds(i * c, c), :], sem.at[i]
    )


def _out_copy(scr, probs_hbm, sem, i, c):
    return pltpu.make_async_copy(
        scr.at[pl.ds(i * c, c), :], probs_hbm.at[pl.ds(i * c, c), :], sem
    )


def _sampler_body(x_hbm, ids_ref, probs_hbm, scr, m_ref, s_ref, a_ref, insem, outsem):
    v, n = scr.shape
    c = _CHUNK
    ns = v // c
    k = _CHAINS
    d = c // k

    for j in range(ns):
        _slab_copy(x_hbm, scr, insem, j, c).start()

    m_ref[...] = jnp.full((1, n), -jnp.inf, jnp.float32)
    s_ref[...] = jnp.zeros((1, n), jnp.float32)
    a_ref[...] = jnp.zeros((1, n), jnp.int32)

    def stats_step(i, carry):
        _slab_copy(x_hbm, scr, insem, i, c).wait()
        xr = scr[pl.ds(i * c, c), :].reshape(k, d, n)
        run_m = m_ref[...]
        pm = jnp.max(xr, axis=1)
        cmax = jnp.max(pm, axis=0, keepdims=True)
        row = lax.broadcasted_iota(jnp.int32, (k, d, n), 1)
        parg = jnp.min(jnp.where(xr == cmax[None], row, v), axis=1)
        offs = lax.broadcasted_iota(jnp.int32, (k, 1), 0) * d
        carg = jnp.min(parg + offs, axis=0, keepdims=True) + i * c
        nmax = jnp.maximum(run_m, cmax)
        bm = nmax * _LOG2E
        ps = jnp.sum(jnp.exp2(xr * _LOG2E - bm[None]), axis=1)
        csum = jnp.sum(ps, axis=0, keepdims=True)
        s_ref[...] = s_ref[...] * jnp.exp2(run_m * _LOG2E - bm) + csum
        a_ref[...] = jnp.where(cmax > run_m, carg, a_ref[...])
        m_ref[...] = nmax
        return carry

    lax.fori_loop(0, ns, stats_step, 0, unroll=False)

    b = m_ref[...] * _LOG2E + jnp.log2(s_ref[...])

    def write_step(i, carry):
        x = scr[pl.ds(i * c, c), :]
        scr[pl.ds(i * c, c), :] = jnp.exp2(x * _LOG2E - b)
        _out_copy(scr, probs_hbm, outsem, i, c).start()
        return carry

    lax.fori_loop(0, ns, write_step, 0, unroll=False)

    def drain_step(i, carry):
        _out_copy(scr, probs_hbm, outsem, i, c).wait()
        return carry

    lax.fori_loop(0, ns, drain_step, 0, unroll=False)
    ids_ref[...] = a_ref[...]


def kernel(logits):
    n, v = logits.shape
    x_t = logits.T
    ids, probs_t = pl.pallas_call(
        _sampler_body,
        in_specs=[pl.BlockSpec(memory_space=pl.ANY)],
        out_specs=[
            pl.BlockSpec((1, n), lambda: (0, 0)),
            pl.BlockSpec(memory_space=pl.ANY),
        ],
        out_shape=[
            jax.ShapeDtypeStruct((1, n), jnp.int32),
            jax.ShapeDtypeStruct((v, n), jnp.float32),
        ],
        scratch_shapes=[
            pltpu.VMEM((v, n), jnp.float32),
            pltpu.VMEM((1, n), jnp.float32),
            pltpu.VMEM((1, n), jnp.float32),
            pltpu.VMEM((1, n), jnp.int32),
            pltpu.SemaphoreType.DMA((v // _CHUNK,)),
            pltpu.SemaphoreType.DMA,
        ],
    )(x_t)
    return (ids.reshape(n), probs_t.T)
